# baseline (speedup 1.0000x reference)
import jax
import jax.numpy as jnp
from jax import lax
from jax.experimental import pallas as pl
from jax.experimental.pallas import tpu as pltpu
from jax.experimental.pallas import tpu_sc as plsc

_NC = 2
_NS = 16
_NW = _NC * _NS
_L = 16

_V = 1000000
_D = 32
_B = 16384 * 26
_Q = 1 << 18
_QB = _Q // 128
_NVB = (_V + 127) // 128

_IPR = 128
_CR = 4
_CHUNK = _CR * _IPR
_NBUF = 2
_PER_W = _B // _NW
_IDX_ROWS_W = _PER_W // _IPR


_TCB = 2048


def _tc_repack_body(w0, w1, w2, w3, out_ref):
  eye = jnp.eye(_D, dtype=jnp.float32)
  for k, wk in enumerate((w0, w1, w2, w3)):
    out_ref[:, k * _D:(k + 1) * _D] = lax.dot_general(
        wk[...], eye, (((0,), (0,)), ((), ())),
        precision=lax.Precision.HIGHEST,
        preferred_element_type=jnp.float32)


def _tc_repack(wt):
  nvb = (_V + _TCB - 1) // _TCB

  def spec(k):
    if k * (_Q // _TCB) + (_Q // _TCB) <= nvb:
      return pl.BlockSpec((_D, _TCB),
                          lambda r, _k=k: (0, r + _k * (_Q // _TCB)))
    return pl.BlockSpec(
        (_D, _TCB),
        lambda r, _k=k: (0, jnp.minimum(r + _k * (_Q // _TCB), nvb - 1)))
  return pl.pallas_call(
      _tc_repack_body,
      grid=(_Q // _TCB,),
      in_specs=[spec(0), spec(1), spec(2), spec(3)],
      out_specs=pl.BlockSpec((_TCB, 4 * _D), lambda r: (r, 0)),
      out_shape=jax.ShapeDtypeStruct((_Q, 4 * _D), jnp.float32),
  )(wt, wt, wt, wt)


def _gather_body(table_hbm, idx_hbm, out_hbm, idx_v, ivs_v, rows_v, idx_sem,
                 gat_sem, out_sem):
  wid = lax.axis_index("s") * _NC + lax.axis_index("c")
  row0 = wid * _IDX_ROWS_W
  n_chunks = _IDX_ROWS_W // _CR
  lanes = lax.iota(jnp.int32, _L)

  def idx_copy(g, q):
    return pltpu.make_async_copy(
        idx_hbm.at[pl.ds(row0 + g * _CR, _CR)], idx_v.at[q], idx_sem.at[q])

  def out_copy(g, q):
    return pltpu.make_async_copy(
        rows_v.at[q], out_hbm.at[pl.ds((row0 + g * _CR) * _IPR, _CHUNK)],
        out_sem.at[q])

  def gather_copies(q):
    return [
        pltpu.make_async_copy(
            table_hbm.at[ivs_v.at[q].at[j]],
            rows_v.at[q].at[pl.ds(j * _IPR, _IPR)],
            gat_sem.at[q],
        )
        for j in range(_CR)
    ]

  def remap(q):
    for j in range(_CR):
      jv = jnp.full((_L,), j, jnp.int32)
      for k in range(_IPR // _L):
        col = lanes + (k * _L)
        v = plsc.load_gather(idx_v.at[q], [jv, col])
        r = jnp.bitwise_and(v, jnp.full((_L,), _Q - 1, jnp.int32)) * 4 + \
            lax.shift_right_logical(v, jnp.full((_L,), 18, jnp.int32))
        plsc.store_scatter(ivs_v.at[q], [jv, col], r)

  idx_copy(0, 0).start()
  idx_copy(1, 1).start()
  out_copy(0, 0).start()
  out_copy(1, 1).start()
  idx_copy(0, 0).wait()
  remap(0)
  for cp in gather_copies(0):
    cp.start()

  def chunk_pair(t, carry):
    for q in range(_NBUF):
      g = t * _NBUF + q
      qn = 1 - q
      gn = jnp.minimum(g + 1, n_chunks - 1)
      idx_copy(gn, qn).wait()
      remap(qn)
      out_copy(gn, qn).wait()
      for cp in gather_copies(qn):
        cp.start()
      for cp in gather_copies(q):
        cp.wait()
      out_copy(g, q).start()
      idx_copy(jnp.minimum(g + 2, n_chunks - 1), q).start()
    return carry

  lax.fori_loop(0, n_chunks // _NBUF, chunk_pair, 0)

  for cp in gather_copies(0):
    cp.wait()
  out_copy(n_chunks - 2, 0).wait()
  out_copy(n_chunks - 1, 1).wait()
  idx_copy(n_chunks - 1, 1).wait()


@jax.jit
def _embed(input, weight):
  wlin = _tc_repack(weight.T).reshape(4 * _Q, _D)
  idx2d = input.reshape(_B // _IPR, _IPR)
  mesh = plsc.VectorSubcoreMesh(core_axis_name="c", subcore_axis_name="s")
  return pl.kernel(
      _gather_body,
      out_type=jax.ShapeDtypeStruct((_B, _D), jnp.float32),
      mesh=mesh,
      scratch_types=[
          pltpu.VMEM((_NBUF, _CR, _IPR), jnp.int32),
          pltpu.VMEM((_NBUF, _CR, _IPR), jnp.int32),
          pltpu.VMEM((_NBUF, _CHUNK, _D), jnp.float32),
          pltpu.SemaphoreType.DMA((_NBUF,)),
          pltpu.SemaphoreType.DMA((_NBUF,)),
          pltpu.SemaphoreType.DMA((_NBUF,)),
      ],
      compiler_params=pltpu.CompilerParams(use_tc_tiling_on_sc=False,
                                           needs_layout_passes=False),
  )(wlin, idx2d)


def kernel(input, weight):
  out = _embed(input, weight)
  return out.reshape(input.shape + (weight.shape[1],))


# speedup vs baseline: 2.0460x; 2.0460x over previous
import jax
import jax.numpy as jnp
from jax import lax
from jax.experimental import pallas as pl
from jax.experimental.pallas import tpu as pltpu
from jax.experimental.pallas import tpu_sc as plsc

_NC = 2
_NS = 16
_NW = _NC * _NS
_L = 16

_V = 1000000
_D = 32
_B = 16384 * 26
_Q = 1 << 18
_QB = _Q // 128
_NVB = (_V + 127) // 128

_IPR = 128
_CR = 4
_CHUNK = _CR * _IPR
_NBUF = 2
_PER_W = _B // _NW
_IDX_ROWS_W = _PER_W // _IPR


_TCB = 2048


def _tc_repack_body(w0, w1, w2, w3, out_ref):
  for k, wk in enumerate((w0, w1, w2, w3)):
    out_ref[:, k * _D:(k + 1) * _D] = wk[...].T


def _tc_repack(wt):
  nvb = (_V + _TCB - 1) // _TCB

  def spec(k):
    if k * (_Q // _TCB) + (_Q // _TCB) <= nvb:
      return pl.BlockSpec((_D, _TCB),
                          lambda r, _k=k: (0, r + _k * (_Q // _TCB)))
    return pl.BlockSpec(
        (_D, _TCB),
        lambda r, _k=k: (0, jnp.minimum(r + _k * (_Q // _TCB), nvb - 1)))
  return pl.pallas_call(
      _tc_repack_body,
      grid=(_Q // _TCB,),
      in_specs=[spec(0), spec(1), spec(2), spec(3)],
      out_specs=pl.BlockSpec((_TCB, 4 * _D), lambda r: (r, 0)),
      out_shape=jax.ShapeDtypeStruct((_Q, 4 * _D), jnp.float32),
  )(wt, wt, wt, wt)


_NCH = _PER_W // _CHUNK
_B2 = 26
_OUT_RPB1 = 32


def _store_runs(g):
  runs = []
  off = 0
  base = _CHUNK * g
  while off < _CHUNK:
    b1l, b2 = divmod(base + off, _B2)
    n = min(_B2 - b2, _CHUNK - off)
    runs.append((off, _OUT_RPB1 * b1l + b2, n))
    off += n
  return runs


def _gather_body(table_hbm, idx_hbm, out_hbm, idx_all, ivs_all, rows_v,
                 idx_sem, gat_sem, out_sem):
  wid = lax.axis_index("s") * _NC + lax.axis_index("c")
  row0 = wid * _IDX_ROWS_W
  wrow0 = wid * (_PER_W // _B2) * _OUT_RPB1
  lanes = lax.iota(jnp.int32, _L)

  pltpu.make_async_copy(idx_hbm.at[pl.ds(row0, _IDX_ROWS_W)], idx_all,
                        idx_sem).start()
  pltpu.make_async_copy(idx_hbm.at[pl.ds(row0, _IDX_ROWS_W)], idx_all,
                        idx_sem).wait()

  def remap_row(r, carry):
    rv = jnp.full((_L,), r, jnp.int32)
    for k in range(_IPR // _L):
      col = lanes + (k * _L)
      v = plsc.load_gather(idx_all, [rv, col])
      rr = jnp.bitwise_and(v, jnp.full((_L,), _Q - 1, jnp.int32)) * 4 + \
          lax.shift_right_logical(v, jnp.full((_L,), 18, jnp.int32))
      plsc.store_scatter(ivs_all, [rv, col], rr)
    return carry
  lax.fori_loop(0, _IDX_ROWS_W, remap_row, 0)

  def gathers(g, q):
    return [
        pltpu.make_async_copy(
            table_hbm.at[ivs_all.at[g * _CR + j]],
            rows_v.at[q].at[pl.ds(j * _IPR, _IPR)],
            gat_sem.at[q],
        )
        for j in range(_CR)
    ]

  def stores(g, q):
    return [
        pltpu.make_async_copy(
            rows_v.at[q].at[pl.ds(so, n)],
            out_hbm.at[pl.ds(wrow0 + ro, n), pl.ds(0, _D)],
            out_sem.at[q],
        )
        for so, ro, n in _store_runs(g)
    ]

  for cp in gathers(0, 0):
    cp.start()
  for cp in gathers(1, 1):
    cp.start()
  for g in range(_NCH):
    q = g % 3
    for cp in gathers(g, q):
      cp.wait()
    for cp in stores(g, q):
      cp.start()
    if g + 2 < _NCH:
      if g >= 1:
        for cp in stores(g - 1, (g + 2) % 3):
          cp.wait()
      for cp in gathers(g + 2, (g + 2) % 3):
        cp.start()
  for g in (_NCH - 2, _NCH - 1):
    for cp in stores(g, g % 3):
      cp.wait()


@jax.jit
def _embed(input, weight):
  wlin = _tc_repack(weight.T).reshape(4 * _Q, _D)
  idx2d = input.reshape(_B // _IPR, _IPR)
  mesh = plsc.VectorSubcoreMesh(core_axis_name="c", subcore_axis_name="s")
  out5 = pl.kernel(
      _gather_body,
      out_type=jax.ShapeDtypeStruct((16384 * _OUT_RPB1, 4 * _D), jnp.float32),
      mesh=mesh,
      scratch_types=[
          pltpu.VMEM((_IDX_ROWS_W, _IPR), jnp.int32),
          pltpu.VMEM((_IDX_ROWS_W, _IPR), jnp.int32),
          pltpu.VMEM((3, _CHUNK, _D), jnp.float32),
          pltpu.SemaphoreType.DMA,
          pltpu.SemaphoreType.DMA((3,)),
          pltpu.SemaphoreType.DMA((3,)),
      ],
      compiler_params=pltpu.CompilerParams(use_tc_tiling_on_sc=False,
                                           needs_layout_passes=False),
  )(wlin, idx2d)
  return out5


def kernel(input, weight):
  out5 = _embed(input, weight)
  return out5.reshape(16384, _OUT_RPB1, 4 * _D)[:, :_B2, :_D]


# speedup vs baseline: 2.0931x; 1.0230x over previous
import jax
import jax.numpy as jnp
from jax import lax
from jax.experimental import pallas as pl
from jax.experimental.pallas import tpu as pltpu
from jax.experimental.pallas import tpu_sc as plsc

_NC = 2
_NS = 16
_NW = _NC * _NS
_L = 16

_V = 1000000
_D = 32
_B = 16384 * 26
_Q = 1 << 18
_QB = _Q // 128
_NVB = (_V + 127) // 128

_IPR = 128
_CR = 4
_CHUNK = _CR * _IPR
_NBUF = 2
_PER_W = _B // _NW
_IDX_ROWS_W = _PER_W // _IPR


_TCB = 4096


def _tc_repack_body(w0, w1, w2, w3, out_ref):
  for k, wk in enumerate((w0, w1, w2, w3)):
    out_ref[:, k * _D:(k + 1) * _D] = wk[...].T


def _tc_repack(wt):
  nvb = (_V + _TCB - 1) // _TCB

  def spec(k):
    if k * (_Q // _TCB) + (_Q // _TCB) <= nvb:
      return pl.BlockSpec((_D, _TCB),
                          lambda r, _k=k: (0, r + _k * (_Q // _TCB)))
    return pl.BlockSpec(
        (_D, _TCB),
        lambda r, _k=k: (0, jnp.minimum(r + _k * (_Q // _TCB), nvb - 1)))
  return pl.pallas_call(
      _tc_repack_body,
      grid=(_Q // _TCB,),
      in_specs=[spec(0), spec(1), spec(2), spec(3)],
      out_specs=pl.BlockSpec((_TCB, 4 * _D), lambda r: (r, 0)),
      out_shape=jax.ShapeDtypeStruct((_Q, 4 * _D), jnp.float32),
  )(wt, wt, wt, wt)


_NCH = _PER_W // _CHUNK
_B2 = 26
_OUT_RPB1 = 32


def _store_runs(g):
  runs = []
  off = 0
  base = _CHUNK * g
  while off < _CHUNK:
    b1l, b2 = divmod(base + off, _B2)
    n = min(_B2 - b2, _CHUNK - off)
    runs.append((off, _OUT_RPB1 * b1l + b2, n))
    off += n
  return runs


def _gather_body(table_hbm, idx_hbm, out_hbm, idx_all, ivs_all, rows_v,
                 idx_sem, gat_sem, out_sem):
  wid = lax.axis_index("s") * _NC + lax.axis_index("c")
  row0 = wid * _IDX_ROWS_W
  wrow0 = wid * (_PER_W // _B2) * _OUT_RPB1
  lanes = lax.iota(jnp.int32, _L)

  pltpu.make_async_copy(idx_hbm.at[pl.ds(row0, _IDX_ROWS_W)], idx_all,
                        idx_sem).start()
  pltpu.make_async_copy(idx_hbm.at[pl.ds(row0, _IDX_ROWS_W)], idx_all,
                        idx_sem).wait()

  def remap_row(r, carry):
    rv = jnp.full((_L,), r, jnp.int32)
    for k in range(_IPR // _L):
      col = lanes + (k * _L)
      v = plsc.load_gather(idx_all, [rv, col])
      rr = jnp.bitwise_and(v, jnp.full((_L,), _Q - 1, jnp.int32)) * 4 + \
          lax.shift_right_logical(v, jnp.full((_L,), 18, jnp.int32))
      plsc.store_scatter(ivs_all, [rv, col], rr)
    return carry
  lax.fori_loop(0, _IDX_ROWS_W, remap_row, 0)

  def gathers(g, q):
    return [
        pltpu.make_async_copy(
            table_hbm.at[ivs_all.at[g * _CR + j]],
            rows_v.at[q].at[pl.ds(j * _IPR, _IPR)],
            gat_sem.at[q],
        )
        for j in range(_CR)
    ]

  def stores(g, q):
    return [
        pltpu.make_async_copy(
            rows_v.at[q].at[pl.ds(so, n)],
            out_hbm.at[pl.ds(wrow0 + ro, n), pl.ds(0, _D)],
            out_sem.at[q],
        )
        for so, ro, n in _store_runs(g)
    ]

  for cp in gathers(0, 0):
    cp.start()
  for cp in gathers(1, 1):
    cp.start()
  for g in range(_NCH):
    q = g % 3
    for cp in gathers(g, q):
      cp.wait()
    for cp in stores(g, q):
      cp.start()
    if g + 2 < _NCH:
      if g >= 1:
        for cp in stores(g - 1, (g + 2) % 3):
          cp.wait()
      for cp in gathers(g + 2, (g + 2) % 3):
        cp.start()
  for g in (_NCH - 2, _NCH - 1):
    for cp in stores(g, g % 3):
      cp.wait()


@jax.jit
def _embed(input, weight):
  wlin = _tc_repack(weight.T).reshape(4 * _Q, _D)
  idx2d = input.reshape(_B // _IPR, _IPR)
  mesh = plsc.VectorSubcoreMesh(core_axis_name="c", subcore_axis_name="s")
  out5 = pl.kernel(
      _gather_body,
      out_type=jax.ShapeDtypeStruct((16384 * _OUT_RPB1, 4 * _D), jnp.float32),
      mesh=mesh,
      scratch_types=[
          pltpu.VMEM((_IDX_ROWS_W, _IPR), jnp.int32),
          pltpu.VMEM((_IDX_ROWS_W, _IPR), jnp.int32),
          pltpu.VMEM((3, _CHUNK, _D), jnp.float32),
          pltpu.SemaphoreType.DMA,
          pltpu.SemaphoreType.DMA((3,)),
          pltpu.SemaphoreType.DMA((3,)),
      ],
      compiler_params=pltpu.CompilerParams(use_tc_tiling_on_sc=False,
                                           needs_layout_passes=False),
  )(wlin, idx2d)
  return out5


def kernel(input, weight):
  out5 = _embed(input, weight)
  return out5.reshape(16384, _OUT_RPB1, 4 * _D)[:, :_B2, :_D]
